# baseline (speedup 1.0000x reference)
import functools

import jax
import jax.numpy as jnp
from jax import lax
from jax.experimental import pallas as pl
from jax.experimental.pallas import tpu as pltpu
from jax.experimental.pallas import tpu_sc as plsc

B = 10
NPG = 1000
N = B * NPG
C = 128
L = 3
E = 320000
EPG = E // B
EPS = 1e-5

PS = 1024
NC = 2
NS = 16
GPC = B // NC
IPT = EPG // NS
SLOTC = 128
NICH = 16
SLOTS = NICH * SLOTC
ZROW = 16
ZCH = ZROW * PS
NZCH = PS * PS // ZCH
ZROUND = NZCH // NS


def _leaky(x):
    return jnp.where(x >= 0, x, 0.01 * x)


DUMP = PS * PS - 1


def _sc_body(edge_hbm, zeros_hbm, zeros2_hbm, ones_hbm, onest_hbm, out_hbm,
             shared, idx_v, src_v, dst_v, ones_v, ones_t_v, zero_v, stage_a,
             stage_b, sem_s, sem_r, sem_w, sem_z):
    c = lax.axis_index("c")
    s = lax.axis_index("s")
    stages = (stage_a, stage_b)
    pltpu.sync_copy(ones_hbm, ones_v)
    pltpu.sync_copy(onest_hbm, ones_t_v)
    pltpu.sync_copy(zeros_hbm, zero_v)
    zd = [pltpu.async_copy(zero_v, shared.at[pl.ds((k * NS + s) * ZCH, ZCH)],
                           sem_z) for k in range(ZROUND)]
    for d in zd:
        d.wait()
    plsc.subcore_barrier()
    for i in range(GPC):
        g = c * GPC + i
        w = g * NS + s
        pltpu.sync_copy(edge_hbm.at[0, w, 0], src_v)
        pltpu.sync_copy(edge_hbm.at[1, w, 0], dst_v)
        goff = g * (NPG * PS + NPG)

        def _idx(t, carry):
            sv = src_v[pl.ds(t * 16, 16)]
            dv = dst_v[pl.ds(t * 16, 16)]
            flat = dv * PS + sv - goff
            pos = t * 16 + lax.iota(jnp.int32, 16)
            flat = jnp.where(pos < IPT, flat, DUMP)
            idx_v[(t * 16) // SLOTC, pl.ds((t * 16) % SLOTC, 16)] = flat
            return carry

        lax.fori_loop(0, SLOTS // 16, _idx, 0)
        plsc.subcore_barrier()
        sd = [pltpu.async_copy(ones_t_v if j == NICH - 1 else ones_v,
                               shared.at[idx_v.at[j]], sem_s,
                               add=True) for j in range(NICH)]
        for d in sd:
            d.wait()
        plsc.subcore_barrier()
        wd = [None] * ZROUND
        zd = [None] * ZROUND
        for k in range(ZROUND):
            ch = k * NS + s
            if k >= 2:
                wd[k - 2].wait()
            st = stages[k % 2]

            def _row(r, carry):
                pltpu.async_copy(shared.at[pl.ds(ch * ZCH + r * PS, PS)],
                                 st.at[r], sem_r)
                return carry

            lax.fori_loop(0, ZROW, _row, 0)
            pltpu.make_async_copy(zeros2_hbm, st, sem_r).wait()
            wd[k] = pltpu.async_copy(
                st, out_hbm.at[pl.ds(g * PS + ch * ZROW, ZROW)], sem_w)
            zd[k] = pltpu.async_copy(zero_v, shared.at[pl.ds(ch * ZCH, ZCH)],
                                     sem_z)
        wd[ZROUND - 2].wait()
        wd[ZROUND - 1].wait()
        for d in zd:
            d.wait()
        plsc.subcore_barrier()


def _sc_build_adjacency(edge3):
    mesh = plsc.VectorSubcoreMesh(core_axis_name="c", subcore_axis_name="s",
                                  num_cores=NC, num_subcores=NS)
    zeros = jnp.zeros((ZCH,), jnp.float32)
    zeros2 = jnp.zeros((ZROW, PS), jnp.float32)
    ones = jnp.ones((SLOTC,), jnp.float32)
    ones_tail = (jnp.arange(SLOTC) < (IPT - (NICH - 1) * SLOTC)
                 ).astype(jnp.float32)
    f = pl.kernel(
        _sc_body,
        out_type=jax.ShapeDtypeStruct((B * PS, PS), jnp.float32),
        mesh=mesh,
        scratch_types=[
            pltpu.VMEM_SHARED((PS * PS,), jnp.float32),
            pltpu.VMEM((NICH, SLOTC), jnp.int32),
            pltpu.VMEM((SLOTS,), jnp.int32),
            pltpu.VMEM((SLOTS,), jnp.int32),
            pltpu.VMEM((SLOTC,), jnp.float32),
            pltpu.VMEM((SLOTC,), jnp.float32),
            pltpu.VMEM((ZCH,), jnp.float32),
            pltpu.VMEM((ZROW, PS), jnp.float32),
            pltpu.VMEM((ZROW, PS), jnp.float32),
            pltpu.SemaphoreType.DMA,
            pltpu.SemaphoreType.DMA,
            pltpu.SemaphoreType.DMA,
            pltpu.SemaphoreType.DMA,
        ],
    )
    return f(edge3, zeros, zeros2, ones, ones_tail)


def _tc_body(a_ref, h0_ref, wa_ref, w_ref, gam_ref, bet_ref, out_ref):
    A = a_ref[0]
    h0 = h0_ref[0]
    A_bf = A.astype(jnp.bfloat16)
    deg_in = jnp.sum(A, axis=1, keepdims=True)
    deg_out = jnp.sum(A, axis=0, keepdims=True)
    norm_dst = jax.lax.rsqrt(jnp.maximum(deg_in, 1.0))
    ns_col = jnp.reshape(jax.lax.rsqrt(jnp.maximum(deg_out, 1.0)), (PS, 1))
    wa = wa_ref[0]

    h = h0
    prev0 = None
    for l in range(L):
        hs = (h * ns_col).astype(jnp.bfloat16)
        agg = jnp.dot(A_bf, hs, preferred_element_type=jnp.float32)
        agg = agg * norm_dst
        h2 = jnp.dot(agg, w_ref[l], preferred_element_type=jnp.float32)
        mean = jnp.sum(h2, axis=0, keepdims=True) / NPG
        var = jnp.sum(h2 * h2, axis=0, keepdims=True) / NPG - mean * mean
        hn = (h2 - mean) * jax.lax.rsqrt(var + EPS)
        hn = hn * gam_ref[l][None, :] + bet_ref[l][None, :]
        hn = _leaky(hn)
        if l == 0:
            ro_src = hn
            prev0 = hn
            h = hn + h0
        else:
            h = hn + prev0
            prev0 = hn
            ro_src = h
        wa_l = jax.lax.slice(wa, (l, 0), (l + 1, PS))
        ro = jnp.dot(wa_l, ro_src, preferred_element_type=jnp.float32)
        out_ref[0, :, l * C:(l + 1) * C] = _leaky(ro)


def _tc_forward(A, h0, wa, W, gamma, beta):
    return pl.pallas_call(
        _tc_body,
        grid=(B,),
        in_specs=[
            pl.BlockSpec((1, PS, PS), lambda g: (g, 0, 0)),
            pl.BlockSpec((1, PS, C), lambda g: (g, 0, 0)),
            pl.BlockSpec((1, L, PS), lambda g: (g, 0, 0)),
            pl.BlockSpec((L, C, C), lambda g: (0, 0, 0)),
            pl.BlockSpec((L, C), lambda g: (0, 0)),
            pl.BlockSpec((L, C), lambda g: (0, 0)),
        ],
        out_specs=pl.BlockSpec((1, 1, L * C), lambda g: (g, 0, 0)),
        out_shape=jax.ShapeDtypeStruct((B, 1, L * C), jnp.float32),
    )(A, h0, wa, W, gamma, beta).reshape(B, L * C)


def kernel(node_feats, edge_index, weights, AR_weights, W, gamma, beta):
    e2 = edge_index.astype(jnp.int32).reshape(2, B * NS, IPT)
    e2 = jnp.pad(e2, ((0, 0), (0, 0), (0, SLOTS - IPT)))
    A = _sc_build_adjacency(e2.reshape(2, B * NS, 1, SLOTS))
    A = A.reshape(B, PS, PS)

    h0 = jnp.pad(node_feats.reshape(B, NPG, C), ((0, 0), (0, PS - NPG), (0, 0)))
    wa = (weights.reshape(B, NPG)[:, None, :] * AR_weights[None, :, :]) / NPG
    wa = jnp.pad(wa, ((0, 0), (0, 0), (0, PS - NPG)))
    return _tc_forward(A, h0, wa, W, gamma, beta)


# speedup vs baseline: 1.0123x; 1.0123x over previous
import functools

import jax
import jax.numpy as jnp
from jax import lax
from jax.experimental import pallas as pl
from jax.experimental.pallas import tpu as pltpu
from jax.experimental.pallas import tpu_sc as plsc

B = 10
NPG = 1000
N = B * NPG
C = 128
L = 3
E = 320000
EPG = E // B
EPS = 1e-5

PS = 1024
NC = 2
NS = 16
GPC = B // NC
IPT = EPG // NS
ICH = 125
NICH = IPT // ICH
ZROW = 16
ZCH = ZROW * PS
NZCH = PS * PS // ZCH
ZROUND = NZCH // NS


def _leaky(x):
    return jnp.where(x >= 0, x, 0.01 * x)


def _sc_body(idx_hbm, zeros_hbm, zeros2_hbm, ones_hbm, out_hbm, shared,
             idx_v, ones_v, zero_v, stage_a, stage_b,
             sem_s, sem_r, sem_w, sem_z):
    c = lax.axis_index("c")
    s = lax.axis_index("s")
    stages = (stage_a, stage_b)
    pltpu.sync_copy(ones_hbm, ones_v)
    pltpu.sync_copy(zeros_hbm, zero_v)
    zd = [pltpu.async_copy(zero_v, shared.at[pl.ds((k * NS + s) * ZCH, ZCH)],
                           sem_z) for k in range(ZROUND)]
    for d in zd:
        d.wait()
    plsc.subcore_barrier()
    for i in range(GPC):
        g = c * GPC + i
        pltpu.sync_copy(idx_hbm.at[g, s], idx_v)
        sd = [pltpu.async_copy(ones_v, shared.at[idx_v.at[j]], sem_s,
                               add=True) for j in range(NICH)]
        for d in sd:
            d.wait()
        plsc.subcore_barrier()
        wd = [None] * ZROUND
        zd = [None] * ZROUND
        for k in range(ZROUND):
            ch = k * NS + s
            if k >= 2:
                wd[k - 2].wait()
            st = stages[k % 2]

            def _row(r, carry):
                pltpu.async_copy(shared.at[pl.ds(ch * ZCH + r * PS, PS)],
                                 st.at[r], sem_r)
                return carry

            lax.fori_loop(0, ZROW, _row, 0)
            pltpu.make_async_copy(zeros2_hbm, st, sem_r).wait()
            wd[k] = pltpu.async_copy(
                st, out_hbm.at[pl.ds(g * PS + ch * ZROW, ZROW)], sem_w)
            zd[k] = pltpu.async_copy(zero_v, shared.at[pl.ds(ch * ZCH, ZCH)],
                                     sem_z)
        wd[ZROUND - 2].wait()
        wd[ZROUND - 1].wait()
        for d in zd:
            d.wait()
        plsc.subcore_barrier()


def _sc_build_adjacency(flat_idx):
    mesh = plsc.VectorSubcoreMesh(core_axis_name="c", subcore_axis_name="s",
                                  num_cores=NC, num_subcores=NS)
    zeros = jnp.zeros((ZCH,), jnp.float32)
    zeros2 = jnp.zeros((ZROW, PS), jnp.float32)
    ones = jnp.ones((ICH,), jnp.float32)
    f = pl.kernel(
        _sc_body,
        out_type=jax.ShapeDtypeStruct((B * PS, PS), jnp.float32),
        mesh=mesh,
        scratch_types=[
            pltpu.VMEM_SHARED((PS * PS,), jnp.float32),
            pltpu.VMEM((NICH, ICH), jnp.int32),
            pltpu.VMEM((ICH,), jnp.float32),
            pltpu.VMEM((ZCH,), jnp.float32),
            pltpu.VMEM((ZROW, PS), jnp.float32),
            pltpu.VMEM((ZROW, PS), jnp.float32),
            pltpu.SemaphoreType.DMA,
            pltpu.SemaphoreType.DMA,
            pltpu.SemaphoreType.DMA,
            pltpu.SemaphoreType.DMA,
        ],
    )
    return f(flat_idx, zeros, zeros2, ones)


def _tc_body(a_ref, h0_ref, wa_ref, w_ref, gam_ref, bet_ref, out_ref):
    A = a_ref[0]
    h0 = h0_ref[0]
    A_bf = A.astype(jnp.bfloat16)
    deg_in = jnp.sum(A, axis=1, keepdims=True)
    deg_out = jnp.sum(A, axis=0, keepdims=True)
    norm_dst = jax.lax.rsqrt(jnp.maximum(deg_in, 1.0))
    ns_col = jnp.reshape(jax.lax.rsqrt(jnp.maximum(deg_out, 1.0)), (PS, 1))
    wa = wa_ref[0]

    h = h0
    prev0 = None
    for l in range(L):
        hs = (h * ns_col).astype(jnp.bfloat16)
        agg = jnp.dot(A_bf, hs, preferred_element_type=jnp.float32)
        agg = agg * norm_dst
        h2 = jnp.dot(agg, w_ref[l], preferred_element_type=jnp.float32)
        mean = jnp.sum(h2, axis=0, keepdims=True) / NPG
        var = jnp.sum(h2 * h2, axis=0, keepdims=True) / NPG - mean * mean
        hn = (h2 - mean) * jax.lax.rsqrt(var + EPS)
        hn = hn * gam_ref[l][None, :] + bet_ref[l][None, :]
        hn = _leaky(hn)
        if l == 0:
            ro_src = hn
            prev0 = hn
            h = hn + h0
        else:
            h = hn + prev0
            prev0 = hn
            ro_src = h
        wa_l = jax.lax.slice(wa, (l, 0), (l + 1, PS))
        ro = jnp.dot(wa_l, ro_src, preferred_element_type=jnp.float32)
        out_ref[0, :, l * C:(l + 1) * C] = _leaky(ro)


def _tc_forward(A, h0, wa, W, gamma, beta):
    return pl.pallas_call(
        _tc_body,
        grid=(B,),
        in_specs=[
            pl.BlockSpec((1, PS, PS), lambda g: (g, 0, 0)),
            pl.BlockSpec((1, PS, C), lambda g: (g, 0, 0)),
            pl.BlockSpec((1, L, PS), lambda g: (g, 0, 0)),
            pl.BlockSpec((L, C, C), lambda g: (0, 0, 0)),
            pl.BlockSpec((L, C), lambda g: (0, 0)),
            pl.BlockSpec((L, C), lambda g: (0, 0)),
        ],
        out_specs=pl.BlockSpec((1, 1, L * C), lambda g: (g, 0, 0)),
        out_shape=jax.ShapeDtypeStruct((B, 1, L * C), jnp.float32),
    )(A, h0, wa, W, gamma, beta).reshape(B, L * C)


def kernel(node_feats, edge_index, weights, AR_weights, W, gamma, beta):
    src = edge_index[0].astype(jnp.int32).reshape(B, EPG)
    dst = edge_index[1].astype(jnp.int32).reshape(B, EPG)
    g_off = jnp.arange(B, dtype=jnp.int32)[:, None] * (NPG * PS + NPG)
    flat = dst * PS + src - g_off
    A = _sc_build_adjacency(flat.reshape(B, NS, NICH, ICH))
    A = A.reshape(B, PS, PS)

    h0 = jnp.pad(node_feats.reshape(B, NPG, C), ((0, 0), (0, PS - NPG), (0, 0)))
    wa = (weights.reshape(B, NPG)[:, None, :] * AR_weights[None, :, :]) / NPG
    wa = jnp.pad(wa, ((0, 0), (0, 0), (0, PS - NPG)))
    return _tc_forward(A, h0, wa, W, gamma, beta)
